# baseline (speedup 1.0000x reference)
import functools
import jax
import jax.numpy as jnp
from jax import lax
from jax.experimental import pallas as pl
from jax.experimental.pallas import tpu as pltpu, tpu_sc as plsc

N = 10000
E = 160000
FP = 32
BD = 16
W48 = 48

NC = 2
NS = 16
NW = NC * NS
CH = 128
CPT = 40
EP = NW * CPT * CH
EPT = CPT * CH
NPAD = 10240
ROWS_PER_TILE = NPAD // NS

EH = EP // 2
CPTH = 20
EPTH = CPTH * CH

_SC_PARAMS = dict(needs_layout_passes=False, use_tc_tiling_on_sc=False)


GSUB = 10


def _gather_body(atom_hbm, atomT_hbm, w1rep_hbm, nbr2_hbm, srcf_hbm,
                 na_hbm, s1src_hbm,
                 idx_n0, idx_n1, src_v, rows0, rows1, s1_v, atT_v, w1_v,
                 out_s, s1_sh, gsem, wsem):
    sid = lax.axis_index("s")
    wid = sid * NC + lax.axis_index("c")
    c0 = wid * CPTH
    e0 = wid * EPTH

    r0 = sid * ROWS_PER_TILE
    pltpu.sync_copy(atomT_hbm.at[:, pl.ds(r0, ROWS_PER_TILE)], atT_v)
    pltpu.sync_copy(w1rep_hbm, w1_v)
    pltpu.sync_copy(srcf_hbm.at[pl.ds(e0, EPTH)], src_v)

    def s1body(g, carry):
        acc = jnp.zeros((16,), jnp.float32)
        for d in range(FP):
            acc = acc + atT_v[d, pl.ds(16 * g, 16)] * w1_v[d, pl.ds(0, 16)]
        acc = acc + w1_v[FP, pl.ds(0, 16)]
        s1_v[pl.ds(r0 + 16 * g, 16)] = acc
        return carry

    lax.fori_loop(0, ROWS_PER_TILE // 16, s1body, 0)
    pltpu.sync_copy(s1_v.at[pl.ds(r0, ROWS_PER_TILE)],
                    s1_sh.at[pl.ds(r0, ROWS_PER_TILE)])
    plsc.subcore_barrier()
    pltpu.sync_copy(s1_sh, s1_v)

    nss = CPTH // GSUB
    idx_bufs = [idx_n0, idx_n1]
    row_bufs = [rows0, rows1]
    pltpu.sync_copy(nbr2_hbm.at[pl.ds(c0, GSUB)], idx_n0)
    wbs = []
    for ss in range(nss):
        idx_b = idx_bufs[ss % 2]
        rows_b = row_bufs[ss % 2]
        if ss >= 2:
            wbs[ss - 2].wait()
        copies = [
            pltpu.async_copy(atom_hbm.at[idx_b.at[j]],
                             rows_b.at[pl.ds(CH * j, CH)], gsem)
            for j in range(GSUB)
        ]
        if ss + 1 < nss:
            pltpu.sync_copy(nbr2_hbm.at[pl.ds(c0 + (ss + 1) * GSUB, GSUB)],
                            idx_bufs[(ss + 1) % 2])
        if ss == 0:
            def sbody(t, carry):
                idx = src_v[pl.ds(16 * t, 16)]
                out_s[pl.ds(16 * t, 16)] = plsc.load_gather(s1_v, [idx])
                return carry

            lax.fori_loop(0, EPTH // 16, sbody, 0)
        for c in copies:
            c.wait()
        wbs.append(
            pltpu.async_copy(rows_b,
                             na_hbm.at[pl.ds(e0 + ss * GSUB * CH, GSUB * CH)],
                             wsem))
    pltpu.sync_copy(out_s, s1src_hbm.at[pl.ds(e0, EPTH)])
    wbs[nss - 2].wait()
    wbs[nss - 1].wait()


def _run_gather(atom, atomT, w1rep, nbr2, srcf):
    mesh = plsc.VectorSubcoreMesh(core_axis_name="c", subcore_axis_name="s")
    f = functools.partial(
        pl.kernel, _gather_body, mesh=mesh,
        compiler_params=pltpu.CompilerParams(**_SC_PARAMS),
        out_type=(jax.ShapeDtypeStruct((EH, FP), jnp.float32),
                  jax.ShapeDtypeStruct((EH,), jnp.float32)),
        scratch_types=[
            pltpu.VMEM((GSUB, CH), jnp.int32),
            pltpu.VMEM((GSUB, CH), jnp.int32),
            pltpu.VMEM((EPTH,), jnp.int32),
            pltpu.VMEM((GSUB * CH, FP), jnp.float32),
            pltpu.VMEM((GSUB * CH, FP), jnp.float32),
            pltpu.VMEM((NPAD,), jnp.float32),
            pltpu.VMEM((FP, ROWS_PER_TILE), jnp.float32),
            pltpu.VMEM((FP + 1, 16), jnp.float32),
            pltpu.VMEM((EPTH,), jnp.float32),
            pltpu.VMEM_SHARED((NPAD,), jnp.float32),
            pltpu.SemaphoreType.DMA,
            pltpu.SemaphoreType.DMA,
        ],
    )
    return f()(atom, atomT, w1rep, nbr2, srcf)


T_EDGE = 1280


def _edge_body(bond_ref, na_ref, s1g_ref, W2_ref, b2_ref, SC_ref,
               ba_ref, out_ref):
    H2 = jnp.maximum(
        jnp.dot(bond_ref[...], W2_ref[...],
                preferred_element_type=jnp.float32) + b2_ref[...], 0.0)
    na = na_ref[...]
    nat = jnp.concatenate([na] * FP, axis=1)
    Z = jnp.dot(nat * H2, SC_ref[...], preferred_element_type=jnp.float32)
    atend = Z[:, :FP] + ba_ref[...]
    x = s1g_ref[...] + Z[:, FP:FP + 1]
    score = jnp.where(x >= 0.0, x, 0.01 * x)
    xs = jnp.exp(score)
    out_ref[...] = jnp.concatenate(
        [xs * atend, xs, jnp.zeros((T_EDGE, W48 - FP - 1), jnp.float32)], axis=1)


def _run_edge(bond_h, na, s1src, W2, b2, SC_, ba, grid_real):
    return pl.pallas_call(
        _edge_body,
        grid=(EH // T_EDGE,),
        in_specs=[
            pl.BlockSpec((T_EDGE, BD), lambda i: (jnp.minimum(i, grid_real), 0)),
            pl.BlockSpec((T_EDGE, FP), lambda i: (i, 0)),
            pl.BlockSpec((T_EDGE, 1), lambda i: (i, 0)),
            pl.BlockSpec((BD, FP * FP), lambda i: (0, 0)),
            pl.BlockSpec((1, FP * FP), lambda i: (0, 0)),
            pl.BlockSpec((FP * FP, FP + 1), lambda i: (0, 0)),
            pl.BlockSpec((1, FP), lambda i: (0, 0)),
        ],
        out_specs=pl.BlockSpec((T_EDGE, W48), lambda i: (i, 0)),
        out_shape=jax.ShapeDtypeStruct((EH, W48), jnp.float32),
    )(bond_h, na, s1src, W2, b2, SC_, ba)


SSUB = 5


def _scatter_body(wat_hbm, src2h_hbm, zeros_hbm, part_hbm,
                  idx_v0, idx_v1, rows_v0, rows_v1, acc_sh, sem):
    cid = lax.axis_index("c")
    sid = lax.axis_index("s")
    wid = sid * NC + cid
    c0 = wid * CPTH
    e0 = wid * EPTH

    pltpu.sync_copy(zeros_hbm,
                    acc_sh.at[pl.ds(sid * ROWS_PER_TILE, ROWS_PER_TILE)])
    plsc.subcore_barrier()

    idx_bufs = [idx_v0, idx_v1]
    row_bufs = [rows_v0, rows_v1]
    nss = CPTH // SSUB
    pltpu.sync_copy(src2h_hbm.at[pl.ds(c0, SSUB)], idx_v0)
    pltpu.sync_copy(wat_hbm.at[pl.ds(e0, SSUB * CH)], rows_v0)
    for ss in range(nss):
        idx_b = idx_bufs[ss % 2]
        rows_b = row_bufs[ss % 2]
        copies = [
            pltpu.async_copy(rows_b.at[pl.ds(CH * j, CH)],
                             acc_sh.at[idx_b.at[j]], sem, add=True)
            for j in range(SSUB)
        ]
        if ss + 1 < nss:
            pltpu.sync_copy(src2h_hbm.at[pl.ds(c0 + (ss + 1) * SSUB, SSUB)],
                            idx_bufs[(ss + 1) % 2])
            pltpu.sync_copy(
                wat_hbm.at[pl.ds(e0 + (ss + 1) * SSUB * CH, SSUB * CH)],
                row_bufs[(ss + 1) % 2])
        for c in copies:
            c.wait()

    plsc.subcore_barrier()
    pltpu.sync_copy(acc_sh.at[pl.ds(sid * ROWS_PER_TILE, ROWS_PER_TILE)],
                    part_hbm.at[cid, pl.ds(sid * ROWS_PER_TILE, ROWS_PER_TILE)])


def _run_scatter(wat_h, src2h, zeros_tile):
    mesh = plsc.VectorSubcoreMesh(core_axis_name="c", subcore_axis_name="s")
    f = functools.partial(
        pl.kernel, _scatter_body, mesh=mesh,
        compiler_params=pltpu.CompilerParams(**_SC_PARAMS),
        out_type=jax.ShapeDtypeStruct((NC, NPAD, W48), jnp.float32),
        scratch_types=[
            pltpu.VMEM((SSUB, CH), jnp.int32),
            pltpu.VMEM((SSUB, CH), jnp.int32),
            pltpu.VMEM((SSUB * CH, W48), jnp.float32),
            pltpu.VMEM((SSUB * CH, W48), jnp.float32),
            pltpu.VMEM_SHARED((NPAD, W48), jnp.float32),
            pltpu.SemaphoreType.DMA,
        ],
    )
    return f()(wat_h, src2h, zeros_tile)


def _node_body(part_ref, partb_ref, atom_ref, WihT_ref, WhhT_ref, bih_ref,
               bhh_ref, out_ref):
    Ssum = (part_ref[0, :N, :] + part_ref[1, :N, :]
            + partb_ref[0, :N, :] + partb_ref[1, :N, :])
    num = Ssum[:, :FP]
    den = Ssum[:, FP:FP + 1]
    ctx = num / (den + 1e-8)
    ctx = jnp.where(ctx > 0.0, ctx, jnp.exp(jnp.minimum(ctx, 0.0)) - 1.0)
    atom = atom_ref[...]
    gi = jnp.dot(ctx, WihT_ref[...], preferred_element_type=jnp.float32) + bih_ref[...]
    gh = jnp.dot(atom, WhhT_ref[...], preferred_element_type=jnp.float32) + bhh_ref[...]
    r = jax.nn.sigmoid(gi[:, :FP] + gh[:, :FP])
    z = jax.nn.sigmoid(gi[:, FP:2 * FP] + gh[:, FP:2 * FP])
    n = jnp.tanh(gi[:, 2 * FP:] + r * gh[:, 2 * FP:])
    out_ref[...] = (1.0 - z) * n + z * atom


def _run_node(part, partb, atom, WihT, WhhT, bih, bhh):
    return pl.pallas_call(
        _node_body,
        out_shape=jax.ShapeDtypeStruct((N, FP), jnp.float32),
    )(part, partb, atom, WihT, WhhT, bih, bhh)


def kernel(atom, bond_index, bond, enc_W, enc_b, enc_g, enc_bt,
           align_W, align_b, att_W, att_b, att_g, att_bt,
           gru_Wih, gru_Whh, gru_bih, gru_bhh):
    npad_e = EP - E
    src = jnp.concatenate([bond_index[:, 0],
                           jnp.full((npad_e,), N, jnp.int32)])
    nbr = jnp.concatenate([bond_index[:, 1],
                           jnp.zeros((npad_e,), jnp.int32)])
    nbr2 = nbr.reshape(EP // CH, CH)
    src2 = src.reshape(EP // CH, CH)

    k = 1.0 / jnp.sqrt(1.0 + 1e-6)
    kg = enc_g * k
    Wf = enc_W * kg[None, :]
    bf = enc_b * kg + enc_bt
    j2 = jnp.arange(FP * FP)
    p = FP * (j2 % FP) + j2 // FP
    W2 = Wf[:, p]
    b2 = bf[p][None, :]
    kga = att_g * k
    Wa = att_W * kga[None, :]
    ba = (att_b * kga + att_bt)[None, :]
    w1 = align_W[:FP]
    w2 = align_W[FP:]
    ab = align_b.reshape(1, 1)
    S = jnp.repeat(jnp.eye(FP, dtype=jnp.float32), FP, axis=0)
    C = jnp.concatenate([Wa, w2], axis=1)
    SC_ = S @ C

    atomT = jnp.concatenate(
        [atom.T, jnp.zeros((FP, NPAD - N), jnp.float32)], axis=1)
    w1rep = jnp.concatenate(
        [jnp.tile(w1, (1, 16)), jnp.full((1, 16), ab[0, 0])])
    na0, s1s0 = _run_gather(atom, atomT, w1rep, nbr2[:EH // CH], src[:EH])
    na1, s1s1 = _run_gather(atom, atomT, w1rep, nbr2[EH // CH:], src[EH:])
    zeros_tile = jnp.zeros((ROWS_PER_TILE, W48), jnp.float32)
    wat0 = _run_edge(bond[:EH], na0, s1s0.reshape(EH, 1), W2, b2, SC_, ba,
                     EH // T_EDGE - 1)
    part0 = _run_scatter(wat0, src2[:EH // CH], zeros_tile)
    wat1 = _run_edge(bond[EH:], na1, s1s1.reshape(EH, 1), W2, b2, SC_, ba,
                     (E - EH) // T_EDGE - 1)
    part1 = _run_scatter(wat1, src2[EH // CH:], zeros_tile)
    update = _run_node(part0, part1, atom, gru_Wih.T, gru_Whh.T,
                       gru_bih[None, :], gru_bhh[None, :])
    return update


# speedup vs baseline: 1.0040x; 1.0040x over previous
import functools
import jax
import jax.numpy as jnp
from jax import lax
from jax.experimental import pallas as pl
from jax.experimental.pallas import tpu as pltpu, tpu_sc as plsc

N = 10000
E = 160000
FP = 32
BD = 16
W48 = 48

NC = 2
NS = 16
NW = NC * NS
CH = 128
CPT = 40
EP = NW * CPT * CH
EPT = CPT * CH
NPAD = 10240
ROWS_PER_TILE = NPAD // NS

EH = EP // 2
CPTH = 20
EPTH = CPTH * CH

_SC_PARAMS = dict(needs_layout_passes=False, use_tc_tiling_on_sc=False)


GSUB = 10


def _gather_body(atom_hbm, atomT_hbm, w1rep_hbm, nbr2_hbm, srcf_hbm,
                 na_hbm, s1src_hbm,
                 idx_n0, idx_n1, src_v, rows0, rows1, s1_v, atT_v, w1_v,
                 out_s, s1_sh, gsem, wsem):
    sid = lax.axis_index("s")
    wid = sid * NC + lax.axis_index("c")
    c0 = wid * CPTH
    e0 = wid * EPTH

    r0 = sid * ROWS_PER_TILE
    pltpu.sync_copy(atomT_hbm.at[:, pl.ds(r0, ROWS_PER_TILE)], atT_v)
    pltpu.sync_copy(w1rep_hbm, w1_v)
    pltpu.sync_copy(srcf_hbm.at[pl.ds(e0, EPTH)], src_v)

    def s1body(g, carry):
        acc = jnp.zeros((16,), jnp.float32)
        for d in range(FP):
            acc = acc + atT_v[d, pl.ds(16 * g, 16)] * w1_v[d, pl.ds(0, 16)]
        acc = acc + w1_v[FP, pl.ds(0, 16)]
        s1_v[pl.ds(r0 + 16 * g, 16)] = acc
        return carry

    lax.fori_loop(0, ROWS_PER_TILE // 16, s1body, 0)
    pltpu.sync_copy(s1_v.at[pl.ds(r0, ROWS_PER_TILE)],
                    s1_sh.at[pl.ds(r0, ROWS_PER_TILE)])
    plsc.subcore_barrier()
    pltpu.sync_copy(s1_sh, s1_v)

    nss = CPTH // GSUB
    idx_bufs = [idx_n0, idx_n1]
    row_bufs = [rows0, rows1]
    pltpu.sync_copy(nbr2_hbm.at[pl.ds(c0, GSUB)], idx_n0)
    wbs = []
    for ss in range(nss):
        idx_b = idx_bufs[ss % 2]
        rows_b = row_bufs[ss % 2]
        if ss >= 2:
            wbs[ss - 2].wait()
        copies = [
            pltpu.async_copy(atom_hbm.at[idx_b.at[j]],
                             rows_b.at[pl.ds(CH * j, CH)], gsem)
            for j in range(GSUB)
        ]
        if ss + 1 < nss:
            pltpu.sync_copy(nbr2_hbm.at[pl.ds(c0 + (ss + 1) * GSUB, GSUB)],
                            idx_bufs[(ss + 1) % 2])
        if ss == 0:
            def sbody(t, carry):
                idx = src_v[pl.ds(16 * t, 16)]
                out_s[pl.ds(16 * t, 16)] = plsc.load_gather(s1_v, [idx])
                return carry

            lax.fori_loop(0, EPTH // 16, sbody, 0)
        for c in copies:
            c.wait()
        wbs.append(
            pltpu.async_copy(rows_b,
                             na_hbm.at[pl.ds(e0 + ss * GSUB * CH, GSUB * CH)],
                             wsem))
    pltpu.sync_copy(out_s, s1src_hbm.at[pl.ds(e0, EPTH)])
    wbs[nss - 2].wait()
    wbs[nss - 1].wait()


def _run_gather(atom, atomT, w1rep, nbr2, srcf):
    mesh = plsc.VectorSubcoreMesh(core_axis_name="c", subcore_axis_name="s")
    f = functools.partial(
        pl.kernel, _gather_body, mesh=mesh,
        compiler_params=pltpu.CompilerParams(**_SC_PARAMS),
        out_type=(jax.ShapeDtypeStruct((EH, FP), jnp.float32),
                  jax.ShapeDtypeStruct((EH,), jnp.float32)),
        scratch_types=[
            pltpu.VMEM((GSUB, CH), jnp.int32),
            pltpu.VMEM((GSUB, CH), jnp.int32),
            pltpu.VMEM((EPTH,), jnp.int32),
            pltpu.VMEM((GSUB * CH, FP), jnp.float32),
            pltpu.VMEM((GSUB * CH, FP), jnp.float32),
            pltpu.VMEM((NPAD,), jnp.float32),
            pltpu.VMEM((FP, ROWS_PER_TILE), jnp.float32),
            pltpu.VMEM((FP + 1, 16), jnp.float32),
            pltpu.VMEM((EPTH,), jnp.float32),
            pltpu.VMEM_SHARED((NPAD,), jnp.float32),
            pltpu.SemaphoreType.DMA,
            pltpu.SemaphoreType.DMA,
        ],
    )
    return f()(atom, atomT, w1rep, nbr2, srcf)


T_EDGE = 1280


def _edge_body(bond_ref, na_ref, s1g_ref, W2_ref, b2_ref, SC_ref,
               ba_ref, out_ref):
    H2 = jnp.maximum(
        jnp.dot(bond_ref[...], W2_ref[...],
                preferred_element_type=jnp.float32) + b2_ref[...], 0.0)
    na = na_ref[...]
    nat = jnp.concatenate([na] * FP, axis=1)
    Z = jnp.dot(nat * H2, SC_ref[...], preferred_element_type=jnp.float32)
    atend = Z[:, :FP] + ba_ref[...]
    x = s1g_ref[...] + Z[:, FP:FP + 1]
    score = jnp.where(x >= 0.0, x, 0.01 * x)
    xs = jnp.exp(score)
    out_ref[...] = jnp.concatenate(
        [xs * atend, xs, jnp.zeros((T_EDGE, W48 - FP - 1), jnp.float32)], axis=1)


def _run_edge(bond_h, na, s1src, W2, b2, SC_, ba, grid_real):
    return pl.pallas_call(
        _edge_body,
        grid=(EH // T_EDGE,),
        in_specs=[
            pl.BlockSpec((T_EDGE, BD), lambda i: (jnp.minimum(i, grid_real), 0)),
            pl.BlockSpec((T_EDGE, FP), lambda i: (i, 0)),
            pl.BlockSpec((T_EDGE, 1), lambda i: (i, 0)),
            pl.BlockSpec((BD, FP * FP), lambda i: (0, 0)),
            pl.BlockSpec((1, FP * FP), lambda i: (0, 0)),
            pl.BlockSpec((FP * FP, FP + 1), lambda i: (0, 0)),
            pl.BlockSpec((1, FP), lambda i: (0, 0)),
        ],
        out_specs=pl.BlockSpec((T_EDGE, W48), lambda i: (i, 0)),
        out_shape=jax.ShapeDtypeStruct((EH, W48), jnp.float32),
    )(bond_h, na, s1src, W2, b2, SC_, ba)


SSUB = 5


def _scatter_body(wat0_hbm, wat1_hbm, src2_hbm, zeros_hbm, part_hbm,
                  idx_v0, idx_v1, rows_v0, rows_v1, acc_sh, sem):
    cid = lax.axis_index("c")
    sid = lax.axis_index("s")
    wid = sid * NC + cid
    c0 = wid * CPT

    pltpu.sync_copy(zeros_hbm,
                    acc_sh.at[pl.ds(sid * ROWS_PER_TILE, ROWS_PER_TILE)])
    plsc.subcore_barrier()

    idx_bufs = [idx_v0, idx_v1]
    row_bufs = [rows_v0, rows_v1]

    def do_half(wat_hbm, e0):
        def run():
            nss = CPT // SSUB
            pltpu.sync_copy(src2_hbm.at[pl.ds(c0, SSUB)], idx_v0)
            pltpu.sync_copy(wat_hbm.at[pl.ds(e0, SSUB * CH)], rows_v0)
            for ss in range(nss):
                idx_b = idx_bufs[ss % 2]
                rows_b = row_bufs[ss % 2]
                copies = [
                    pltpu.async_copy(rows_b.at[pl.ds(CH * j, CH)],
                                     acc_sh.at[idx_b.at[j]], sem, add=True)
                    for j in range(SSUB)
                ]
                if ss + 1 < nss:
                    pltpu.sync_copy(
                        src2_hbm.at[pl.ds(c0 + (ss + 1) * SSUB, SSUB)],
                        idx_bufs[(ss + 1) % 2])
                    pltpu.sync_copy(
                        wat_hbm.at[pl.ds(e0 + (ss + 1) * SSUB * CH, SSUB * CH)],
                        row_bufs[(ss + 1) % 2])
                for c in copies:
                    c.wait()
        return run

    pl.when(wid < NS)(do_half(wat0_hbm, wid * EPT))
    pl.when(wid >= NS)(do_half(wat1_hbm, (wid - NS) * EPT))

    plsc.subcore_barrier()
    pltpu.sync_copy(acc_sh.at[pl.ds(sid * ROWS_PER_TILE, ROWS_PER_TILE)],
                    part_hbm.at[cid, pl.ds(sid * ROWS_PER_TILE, ROWS_PER_TILE)])


def _run_scatter(wat0, wat1, src2, zeros_tile):
    mesh = plsc.VectorSubcoreMesh(core_axis_name="c", subcore_axis_name="s")
    f = functools.partial(
        pl.kernel, _scatter_body, mesh=mesh,
        compiler_params=pltpu.CompilerParams(**_SC_PARAMS),
        out_type=jax.ShapeDtypeStruct((NC, NPAD, W48), jnp.float32),
        scratch_types=[
            pltpu.VMEM((SSUB, CH), jnp.int32),
            pltpu.VMEM((SSUB, CH), jnp.int32),
            pltpu.VMEM((SSUB * CH, W48), jnp.float32),
            pltpu.VMEM((SSUB * CH, W48), jnp.float32),
            pltpu.VMEM_SHARED((NPAD, W48), jnp.float32),
            pltpu.SemaphoreType.DMA,
        ],
    )
    return f()(wat0, wat1, src2, zeros_tile)


def _node_body(part_ref, atom_ref, WihT_ref, WhhT_ref, bih_ref, bhh_ref,
               out_ref):
    Ssum = part_ref[0, :N, :] + part_ref[1, :N, :]
    num = Ssum[:, :FP]
    den = Ssum[:, FP:FP + 1]
    ctx = num / (den + 1e-8)
    ctx = jnp.where(ctx > 0.0, ctx, jnp.exp(jnp.minimum(ctx, 0.0)) - 1.0)
    atom = atom_ref[...]
    gi = jnp.dot(ctx, WihT_ref[...], preferred_element_type=jnp.float32) + bih_ref[...]
    gh = jnp.dot(atom, WhhT_ref[...], preferred_element_type=jnp.float32) + bhh_ref[...]
    r = jax.nn.sigmoid(gi[:, :FP] + gh[:, :FP])
    z = jax.nn.sigmoid(gi[:, FP:2 * FP] + gh[:, FP:2 * FP])
    n = jnp.tanh(gi[:, 2 * FP:] + r * gh[:, 2 * FP:])
    out_ref[...] = (1.0 - z) * n + z * atom


def _run_node(part, atom, WihT, WhhT, bih, bhh):
    return pl.pallas_call(
        _node_body,
        out_shape=jax.ShapeDtypeStruct((N, FP), jnp.float32),
    )(part, atom, WihT, WhhT, bih, bhh)


def kernel(atom, bond_index, bond, enc_W, enc_b, enc_g, enc_bt,
           align_W, align_b, att_W, att_b, att_g, att_bt,
           gru_Wih, gru_Whh, gru_bih, gru_bhh):
    npad_e = EP - E
    src = jnp.concatenate([bond_index[:, 0],
                           jnp.full((npad_e,), N, jnp.int32)])
    nbr = jnp.concatenate([bond_index[:, 1],
                           jnp.zeros((npad_e,), jnp.int32)])
    nbr2 = nbr.reshape(EP // CH, CH)
    src2 = src.reshape(EP // CH, CH)

    k = 1.0 / jnp.sqrt(1.0 + 1e-6)
    kg = enc_g * k
    Wf = enc_W * kg[None, :]
    bf = enc_b * kg + enc_bt
    j2 = jnp.arange(FP * FP)
    p = FP * (j2 % FP) + j2 // FP
    W2 = Wf[:, p]
    b2 = bf[p][None, :]
    kga = att_g * k
    Wa = att_W * kga[None, :]
    ba = (att_b * kga + att_bt)[None, :]
    w1 = align_W[:FP]
    w2 = align_W[FP:]
    ab = align_b.reshape(1, 1)
    S = jnp.repeat(jnp.eye(FP, dtype=jnp.float32), FP, axis=0)
    C = jnp.concatenate([Wa, w2], axis=1)
    SC_ = S @ C

    atomT = jnp.concatenate(
        [atom.T, jnp.zeros((FP, NPAD - N), jnp.float32)], axis=1)
    w1rep = jnp.concatenate(
        [jnp.tile(w1, (1, 16)), jnp.full((1, 16), ab[0, 0])])
    na0, s1s0 = _run_gather(atom, atomT, w1rep, nbr2[:EH // CH], src[:EH])
    na1, s1s1 = _run_gather(atom, atomT, w1rep, nbr2[EH // CH:], src[EH:])
    wat0 = _run_edge(bond[:EH], na0, s1s0.reshape(EH, 1), W2, b2, SC_, ba,
                     EH // T_EDGE - 1)
    wat1 = _run_edge(bond[EH:], na1, s1s1.reshape(EH, 1), W2, b2, SC_, ba,
                     (E - EH) // T_EDGE - 1)
    zeros_tile = jnp.zeros((ROWS_PER_TILE, W48), jnp.float32)
    part = _run_scatter(wat0, wat1, src2, zeros_tile)
    update = _run_node(part, atom, gru_Wih.T, gru_Whh.T,
                       gru_bih[None, :], gru_bhh[None, :])
    return update
